# baseline (speedup 1.0000x reference)
import functools
import math

import jax
import jax.numpy as jnp
import numpy as np
from jax import lax
from jax.experimental import pallas as pl
from jax.experimental.pallas import tpu as pltpu
from jax.experimental.pallas import tpu_sc as plsc

CUTOFF = 5.0


def _gather_z_sc(z, eidx_flat):
    n = z.shape[0]
    te = eidx_flat.shape[0]
    nw = 32
    per = te // nw
    iters = per // 16
    assert per * nw == te and iters * 16 == per

    mesh = plsc.VectorSubcoreMesh(core_axis_name="c", subcore_axis_name="s")

    @functools.partial(
        pl.kernel,
        out_type=jax.ShapeDtypeStruct((te,), jnp.int32),
        mesh=mesh,
        compiler_params=pltpu.CompilerParams(needs_layout_passes=False),
        scratch_types=[
            pltpu.VMEM((n,), jnp.int32),
            pltpu.VMEM((per,), jnp.int32),
            pltpu.VMEM((per,), jnp.int32),
        ],
    )
    def k(z_hbm, idx_hbm, out_hbm, z_v, idx_v, out_v):
        wid = lax.axis_index("s") * 2 + lax.axis_index("c")
        base = wid * per
        pltpu.sync_copy(z_hbm, z_v)
        pltpu.sync_copy(idx_hbm.at[pl.ds(base, per)], idx_v)

        @pl.loop(0, iters)
        def _(i):
            idx = idx_v[pl.ds(i * 16, 16)]
            out_v[pl.ds(i * 16, 16)] = plsc.load_gather(z_v, [idx])

        pltpu.sync_copy(out_v, out_hbm.at[pl.ds(base, per)])

    return k(z, eidx_flat)


def _edge_payload_tc(scal, edge_attr, emb_p, apw, apb,
                     wi, bi, wa, ba, ws, bs, eb, off_blk, nblk):
    e = edge_attr.shape[0]
    h = apw.shape[1]
    nrbf = edge_attr.shape[1]
    assert e % eb == 0
    last_ea_blk = e // eb - 1
    dn = (((0,), (0,)), ((), ()))

    def body(sc_ref, ea_ref, emb_ref, apw_ref, apb_ref,
             wi_ref, bi_ref, wa_ref, ba_ref, ws_ref, bs_ref, out_ref):
        f32 = jnp.float32
        a1 = jnp.dot(emb_ref[...], apw_ref[:h, :], preferred_element_type=f32)
        a2 = jnp.dot(emb_ref[...], apw_ref[h:, :], preferred_element_type=f32)
        ones_r = jnp.ones((1, h), f32)
        s = sc_ref[...]
        bc = lambda r: lax.dot_general(r, ones_r, dn,
                                       preferred_element_type=f32)
        bch = lambda r: lax.dot_general(r, ones_r, dn,
                                        preferred_element_type=f32,
                                        precision=lax.Precision.HIGHEST)
        zrm = bc(s[0:1, :])
        zcm = bc(s[1:2, :])
        cols = lax.broadcasted_iota(jnp.int32, (eb, h), 1).astype(f32)
        ohr = (zrm == cols).astype(f32)
        ohc = (zcm == cols).astype(f32)
        zij = (jnp.dot(ohr, a1, preferred_element_type=f32)
               + jnp.dot(ohc, a2, preferred_element_type=f32)
               + apb_ref[...])
        d = s[2:3, :]
        cval = 0.5 * (jnp.cos(d * (math.pi / CUTOFF)) + 1.0)
        cval = jnp.where(d < CUTOFF, cval, 0.0)
        czm = bch(cval) * zij
        ea = ea_ref[...]
        di = jnp.dot(ea, wi_ref[...], preferred_element_type=f32) + bi_ref[...]
        da = jnp.dot(ea, wa_ref[...], preferred_element_type=f32) + ba_ref[...]
        ds_ = jnp.dot(ea, ws_ref[...], preferred_element_type=f32) + bs_ref[...]
        w_i = di * czm
        w_a = da * czm
        w_s = ds_ * czm
        a0 = bch(s[3:4, :])
        a1v = bch(s[4:5, :])
        a2v = bch(s[5:6, :])
        out_ref[0] = w_i
        out_ref[1] = w_a * a0
        out_ref[2] = w_a * a1v
        out_ref[3] = w_a * a2v
        out_ref[4] = w_s * (a0 * a0)
        out_ref[5] = w_s * (a0 * a1v)
        out_ref[6] = w_s * (a0 * a2v)
        out_ref[7] = w_s * (a1v * a1v)
        out_ref[8] = w_s * (a1v * a2v)
        out_ref[9] = w_s * (a2v * a2v)

    full = lambda shape: pl.BlockSpec(shape, lambda i: tuple(0 for _ in shape))
    return pl.pallas_call(
        body,
        grid=(nblk,),
        in_specs=[
            pl.BlockSpec((6, eb), lambda i: (0, i + off_blk)),
            pl.BlockSpec((eb, nrbf),
                         lambda i: (jnp.minimum(i + off_blk, last_ea_blk), 0)),
            full(emb_p.shape), full(apw.shape), full((1, h)),
            full((nrbf, h)), full((1, h)),
            full((nrbf, h)), full((1, h)),
            full((nrbf, h)), full((1, h)),
        ],
        out_specs=pl.BlockSpec((10, eb, h), lambda i: (0, i, 0)),
        out_shape=jax.ShapeDtypeStruct((10, nblk * eb, h), jnp.float32),
        compiler_params=pltpu.CompilerParams(
            dimension_semantics=("arbitrary",)),
    )(scal, edge_attr, emb_p, apw, apb, wi, bi, wa, ba, ws, bs)


def _scatter_sc(payload, row3, n):
    c, e, h = payload.shape
    nt = 16
    per_tile = e // nt
    nbt, b = row3.shape[1], row3.shape[2]
    assert nbt * b == per_tile and row3.shape[0] == nt
    stripe = n // nt
    zr = 16
    assert stripe % zr == 0 and stripe % 8 == 0
    cpc = c // 2

    mesh = plsc.VectorSubcoreMesh(core_axis_name="c", subcore_axis_name="s")

    @functools.partial(
        pl.kernel,
        out_type=jax.ShapeDtypeStruct((c, n, h), jnp.float32),
        mesh=mesh,
        compiler_params=pltpu.CompilerParams(needs_layout_passes=False),
        scratch_types=(
            [pltpu.VMEM_SHARED((n, h), jnp.float32)]
            + [pltpu.VMEM((b, h), jnp.float32),
               pltpu.VMEM((b, h), jnp.float32),
               pltpu.VMEM((nbt, b), jnp.int32),
               pltpu.VMEM((zr, h), jnp.float32)]
            + [pltpu.SemaphoreType.DMA for _ in range(5)]
        ),
    )
    def k(pay_hbm, row_hbm, out_hbm, acc, pb0, pb1, idx_v, zb,
          sp0, sp1, ss0, ss1, zs):
        core = lax.axis_index("c")
        sub = lax.axis_index("s")
        pbufs = [pb0, pb1]
        psems = [sp0, sp1]
        ssems = [ss0, ss1]

        pltpu.sync_copy(row_hbm.at[sub], idx_v)

        @pl.loop(0, zr)
        def _(r):
            for c16 in range(h // 16):
                zb[r, pl.ds(c16 * 16, 16)] = jnp.zeros((16,), jnp.float32)

        sbase = sub * stripe
        nz = stripe // zr

        def zero_stripe():
            @pl.loop(0, nz)
            def _(t):
                pltpu.async_copy(zb, acc.at[pl.ds(sbase + t * zr, zr)], zs)

            @pl.loop(0, nz)
            def _(t):
                pltpu.make_async_copy(
                    zb, acc.at[pl.ds(sbase, zr)], zs).wait()

        def wait_pay(kb):
            pltpu.make_async_copy(
                pay_hbm.at[0, pl.ds(0, b)], pbufs[kb], psems[kb]).wait()

        def wait_scat(kb):
            pltpu.make_async_copy(
                pbufs[kb], acc.at[idx_v.at[0]], ssems[kb]).wait()

        def pref(chunk, bi_, kb):
            base = sub * per_tile + bi_ * b
            pltpu.async_copy(pay_hbm.at[chunk, pl.ds(base, b)],
                             pbufs[kb], psems[kb])

        zero_stripe()
        for j in range(cpc):
            chunk = core * cpc + j
            plsc.subcore_barrier()
            pref(chunk, 0, 0)

            @pl.loop(0, nbt, step=2)
            def _(b0):
                for kb in range(2):
                    bi_ = b0 + kb
                    ot = 1 - kb
                    wait_pay(kb)
                    pltpu.async_copy(pbufs[kb], acc.at[idx_v.at[bi_]],
                                     ssems[kb], add=True)

                    @pl.when(bi_ >= 1)
                    def _():
                        wait_scat(ot)

                    @pl.when(bi_ + 1 < nbt)
                    def _():
                        pref(chunk, bi_ + 1, ot)

            wait_scat((nbt - 1) % 2)
            plsc.subcore_barrier()
            pltpu.sync_copy(acc.at[pl.ds(sbase, stripe)],
                            out_hbm.at[chunk, pl.ds(sbase, stripe)])
            if j < cpc - 1:
                zero_stripe()

    return k(payload, row3)


def _node_tc(moments, moments2, ln_g, ln_b, mlp1_w, mlp1_b, mlp2_wr, mlp2_br,
             lini_w, lina_w, lins_w, nb, n):
    h = moments.shape[2]

    def body(m_ref, m2_ref, g_ref, be_ref, w1_ref, b1_ref, w2_ref, b2_ref,
             li_ref, la_ref, ls_ref, out_ref):
        m = m_ref[...] + m2_ref[...]
        r = m[0]
        p0, p1, p2 = m[1], m[2], m[3]
        q0, q1, q2, q3, q4, q5 = (m[4], m[5], m[6], m[7], m[8], m[9])
        tn = ((r + q0) ** 2 + (r + q3) ** 2 + (r + q5) ** 2
              + 2.0 * (q1 * q1 + p2 * p2)
              + 2.0 * (q2 * q2 + p1 * p1)
              + 2.0 * (q4 * q4 + p0 * p0))
        mu = jnp.mean(tn, axis=1, keepdims=True)
        var = jnp.mean((tn - mu) ** 2, axis=1, keepdims=True)
        hh = (tn - mu) * lax.rsqrt(var + 1e-5) * g_ref[...] + be_ref[...]
        h1 = jnp.dot(hh, w1_ref[...], preferred_element_type=jnp.float32) + b1_ref[...]
        h1 = h1 / (1.0 + jnp.exp(-h1))
        h2 = jnp.dot(h1, w2_ref[...], preferred_element_type=jnp.float32) + b2_ref[...]
        n0 = h2[:, :h]
        n1 = h2[:, h:2 * h]
        n2 = h2[:, 2 * h:]
        dot = lambda x, wref: jnp.dot(x, wref[...], preferred_element_type=jnp.float32)
        ni = dot(r, li_ref) * n0
        np0 = dot(p0, la_ref) * n1
        np1 = dot(p1, la_ref) * n1
        np2 = dot(p2, la_ref) * n1
        nq0 = dot(q0, ls_ref) * n2
        nq1 = dot(q1, ls_ref) * n2
        nq2 = dot(q2, ls_ref) * n2
        nq3 = dot(q3, ls_ref) * n2
        nq4 = dot(q4, ls_ref) * n2
        nq5 = dot(q5, ls_ref) * n2
        out_ref[0] = ni + nq0
        out_ref[1] = nq1 - np2
        out_ref[2] = nq2 + np1
        out_ref[3] = nq1 + np2
        out_ref[4] = ni + nq3
        out_ref[5] = nq4 - np0
        out_ref[6] = nq2 - np1
        out_ref[7] = nq4 + np0
        out_ref[8] = ni + nq5

    full = lambda shape: pl.BlockSpec(shape, lambda i: tuple(0 for _ in shape))
    return pl.pallas_call(
        body,
        grid=(n // nb,),
        in_specs=[
            pl.BlockSpec((10, nb, h), lambda i: (0, i, 0)),
            pl.BlockSpec((10, nb, h), lambda i: (0, i, 0)),
            full((1, h)), full((1, h)),
            full(mlp1_w.shape), full((1, mlp1_w.shape[1])),
            full(mlp2_wr.shape), full((1, mlp2_wr.shape[1])),
            full((h, h)), full((h, h)), full((h, h)),
        ],
        out_specs=pl.BlockSpec((9, nb, h), lambda i: (0, i, 0)),
        out_shape=jax.ShapeDtypeStruct((9, n, h), jnp.float32),
        compiler_params=pltpu.CompilerParams(
            dimension_semantics=("arbitrary",)),
    )(moments, moments2, ln_g, ln_b, mlp1_w, mlp1_b, mlp2_wr, mlp2_br,
      lini_w, lina_w, lins_w)


def kernel(z, edge_index, edge_dist, edge_vec_norm, edge_attr,
           atom_emb, atom_proj_w, atom_proj_b,
           distI_w, distI_b, distA_w, distA_b, distS_w, distS_b,
           linI_w, linA_w, linS_w, ln_g, ln_b,
           mlp1_w, mlp1_b, mlp2_w, mlp2_b):
    n = z.shape[0]
    e = edge_index.shape[1]
    h = atom_emb.shape[1]

    z = z.astype(jnp.int32)

    e_pad = ((e + 4095) // 4096) * 4096
    pe = e_pad - e
    row_i = edge_index[0].astype(jnp.int32)
    col_i = edge_index[1].astype(jnp.int32)
    row_p = jnp.pad(row_i, (0, pe))
    col_p = jnp.pad(col_i, (0, pe))
    row_scat = jnp.pad(row_i, (0, pe), constant_values=n)
    eidx = jnp.concatenate([row_p, col_p])

    zflat = _gather_z_sc(z, eidx)

    emb_p = jnp.pad(atom_emb, ((0, h - atom_emb.shape[0]), (0, 0)))
    scal = jnp.concatenate([
        zflat.reshape(2, e_pad).astype(jnp.float32),
        jnp.pad(jnp.stack([edge_dist, edge_vec_norm[:, 0],
                           edge_vec_norm[:, 1], edge_vec_norm[:, 2]]),
                ((0, 0), (0, pe))),
    ])
    row1 = lambda x: x.reshape(1, -1)

    b = 128
    eb = 1280
    n_pad = ((n + 2047) // 2048) * 2048
    e_half = e_pad // 2
    moments = []
    for s in range(2):
        sl = slice(s * e_half, (s + 1) * e_half)
        pay = _edge_payload_tc(
            scal, edge_attr, emb_p, atom_proj_w, row1(atom_proj_b),
            distI_w, row1(distI_b), distA_w, row1(distA_b),
            distS_w, row1(distS_b), eb=eb,
            off_blk=s * (e_half // eb), nblk=e_half // eb)
        row3 = row_scat[sl].reshape(16, e_half // 16 // b, b)
        moments.append(_scatter_sc(pay, row3, n_pad))

    perm = (3 * np.arange(h)[None, :] + np.arange(3)[:, None]).reshape(-1)
    mlp2_wr = mlp2_w[:, perm]
    mlp2_br = mlp2_b[perm].reshape(1, -1)

    out9 = _node_tc(moments[0], moments[1], row1(ln_g), row1(ln_b),
                    mlp1_w, row1(mlp1_b), mlp2_wr, mlp2_br,
                    linI_w, linA_w, linS_w, nb=1000, n=n)
    return jnp.transpose(out9, (1, 2, 0)).reshape(n, h, 3, 3)


# speedup vs baseline: 1.0326x; 1.0326x over previous
import functools
import math

import jax
import jax.numpy as jnp
import numpy as np
from jax import lax
from jax.experimental import pallas as pl
from jax.experimental.pallas import tpu as pltpu
from jax.experimental.pallas import tpu_sc as plsc

CUTOFF = 5.0


def _gather_z_sc(z, eidx_flat):
    n = z.shape[0]
    te = eidx_flat.shape[0]
    nw = 32
    per = te // nw
    iters = per // 16
    assert per * nw == te and iters * 16 == per

    mesh = plsc.VectorSubcoreMesh(core_axis_name="c", subcore_axis_name="s")

    @functools.partial(
        pl.kernel,
        out_type=jax.ShapeDtypeStruct((te,), jnp.int32),
        mesh=mesh,
        compiler_params=pltpu.CompilerParams(needs_layout_passes=False),
        scratch_types=[
            pltpu.VMEM((n,), jnp.int32),
            pltpu.VMEM((per,), jnp.int32),
            pltpu.VMEM((per,), jnp.int32),
        ],
    )
    def k(z_hbm, idx_hbm, out_hbm, z_v, idx_v, out_v):
        wid = lax.axis_index("s") * 2 + lax.axis_index("c")
        base = wid * per
        pltpu.sync_copy(z_hbm, z_v)
        pltpu.sync_copy(idx_hbm.at[pl.ds(base, per)], idx_v)

        @pl.loop(0, iters)
        def _(i):
            idx = idx_v[pl.ds(i * 16, 16)]
            out_v[pl.ds(i * 16, 16)] = plsc.load_gather(z_v, [idx])

        pltpu.sync_copy(out_v, out_hbm.at[pl.ds(base, per)])

    return k(z, eidx_flat)


def _edge_payload_tc(scal, edge_attr, emb_p, apw, apb,
                     wi, bi, wa, ba, ws, bs, eb, off_blk, nblk):
    e = edge_attr.shape[0]
    h = apw.shape[1]
    nrbf = edge_attr.shape[1]
    assert e % eb == 0
    last_ea_blk = e // eb - 1
    dn = (((0,), (0,)), ((), ()))

    def body(sc_ref, ea_ref, emb_ref, apw_ref, apb_ref,
             wi_ref, bi_ref, wa_ref, ba_ref, ws_ref, bs_ref, out_ref):
        f32 = jnp.float32
        a1 = jnp.dot(emb_ref[...], apw_ref[:h, :], preferred_element_type=f32)
        a2 = jnp.dot(emb_ref[...], apw_ref[h:, :], preferred_element_type=f32)
        ones_r = jnp.ones((1, h), f32)
        s = sc_ref[...]
        bc = lambda r: lax.dot_general(r, ones_r, dn,
                                       preferred_element_type=f32)
        bch = lambda r: lax.dot_general(r, ones_r, dn,
                                        preferred_element_type=f32,
                                        precision=lax.Precision.HIGHEST)
        zrm = bc(s[0:1, :])
        zcm = bc(s[1:2, :])
        cols = lax.broadcasted_iota(jnp.int32, (eb, h), 1).astype(f32)
        ohr = (zrm == cols).astype(f32)
        ohc = (zcm == cols).astype(f32)
        zij = (jnp.dot(ohr, a1, preferred_element_type=f32)
               + jnp.dot(ohc, a2, preferred_element_type=f32)
               + apb_ref[...])
        d = s[2:3, :]
        cval = 0.5 * (jnp.cos(d * (math.pi / CUTOFF)) + 1.0)
        cval = jnp.where(d < CUTOFF, cval, 0.0)
        czm = bch(cval) * zij
        ea = ea_ref[...]
        di = jnp.dot(ea, wi_ref[...], preferred_element_type=f32) + bi_ref[...]
        da = jnp.dot(ea, wa_ref[...], preferred_element_type=f32) + ba_ref[...]
        ds_ = jnp.dot(ea, ws_ref[...], preferred_element_type=f32) + bs_ref[...]
        w_i = di * czm
        w_a = da * czm
        w_s = ds_ * czm
        a0 = bch(s[3:4, :])
        a1v = bch(s[4:5, :])
        a2v = bch(s[5:6, :])
        out_ref[0] = w_i
        out_ref[1] = w_a * a0
        out_ref[2] = w_a * a1v
        out_ref[3] = w_a * a2v
        out_ref[4] = w_s * (a0 * a0)
        out_ref[5] = w_s * (a0 * a1v)
        out_ref[6] = w_s * (a0 * a2v)
        out_ref[7] = w_s * (a1v * a1v)
        out_ref[8] = w_s * (a1v * a2v)
        out_ref[9] = w_s * (a2v * a2v)

    full = lambda shape: pl.BlockSpec(shape, lambda i: tuple(0 for _ in shape))
    return pl.pallas_call(
        body,
        grid=(nblk,),
        in_specs=[
            pl.BlockSpec((6, eb), lambda i: (0, i + off_blk)),
            pl.BlockSpec((eb, nrbf),
                         lambda i: (jnp.minimum(i + off_blk, last_ea_blk), 0)),
            full(emb_p.shape), full(apw.shape), full((1, h)),
            full((nrbf, h)), full((1, h)),
            full((nrbf, h)), full((1, h)),
            full((nrbf, h)), full((1, h)),
        ],
        out_specs=pl.BlockSpec((10, eb, h), lambda i: (0, i, 0)),
        out_shape=jax.ShapeDtypeStruct((10, nblk * eb, h), jnp.float32),
        compiler_params=pltpu.CompilerParams(
            dimension_semantics=("arbitrary",)),
    )(scal, edge_attr, emb_p, apw, apb, wi, bi, wa, ba, ws, bs)


def _scatter_sc(payload, row3, n):
    c, e, h = payload.shape
    nt = 16
    per_tile = e // nt
    nbt, b = row3.shape[1], row3.shape[2]
    assert nbt * b == per_tile and row3.shape[0] == nt
    stripe = n // nt
    zr = 16
    assert stripe % zr == 0 and stripe % 8 == 0
    cpc = c // 2

    mesh = plsc.VectorSubcoreMesh(core_axis_name="c", subcore_axis_name="s")

    nbuf = 4
    assert nbt % nbuf == 0

    @functools.partial(
        pl.kernel,
        out_type=jax.ShapeDtypeStruct((c, n, h), jnp.float32),
        mesh=mesh,
        compiler_params=pltpu.CompilerParams(needs_layout_passes=False),
        scratch_types=(
            [pltpu.VMEM_SHARED((n, h), jnp.float32)]
            + [pltpu.VMEM((b, h), jnp.float32) for _ in range(nbuf)]
            + [pltpu.VMEM((nbt, b), jnp.int32),
               pltpu.VMEM((zr, h), jnp.float32)]
            + [pltpu.SemaphoreType.DMA for _ in range(2 * nbuf + 1)]
        ),
    )
    def k(pay_hbm, row_hbm, out_hbm, acc, pb0, pb1, pb2, pb3, idx_v, zb,
          sp0, sp1, sp2, sp3, ss0, ss1, ss2, ss3, zs):
        core = lax.axis_index("c")
        sub = lax.axis_index("s")
        pbufs = [pb0, pb1, pb2, pb3]
        psems = [sp0, sp1, sp2, sp3]
        ssems = [ss0, ss1, ss2, ss3]

        pltpu.sync_copy(row_hbm.at[sub], idx_v)

        @pl.loop(0, zr)
        def _(r):
            for c16 in range(h // 16):
                zb[r, pl.ds(c16 * 16, 16)] = jnp.zeros((16,), jnp.float32)

        sbase = sub * stripe
        nz = stripe // zr

        def zero_stripe():
            @pl.loop(0, nz)
            def _(t):
                pltpu.async_copy(zb, acc.at[pl.ds(sbase + t * zr, zr)], zs)

            @pl.loop(0, nz)
            def _(t):
                pltpu.make_async_copy(
                    zb, acc.at[pl.ds(sbase, zr)], zs).wait()

        def wait_pay(kb):
            pltpu.make_async_copy(
                pay_hbm.at[0, pl.ds(0, b)], pbufs[kb], psems[kb]).wait()

        def wait_scat(kb):
            pltpu.make_async_copy(
                pbufs[kb], acc.at[idx_v.at[0]], ssems[kb]).wait()

        def pref(chunk, bi_, kb):
            base = sub * per_tile + bi_ * b
            pltpu.async_copy(pay_hbm.at[chunk, pl.ds(base, b)],
                             pbufs[kb], psems[kb])

        zero_stripe()
        for j in range(cpc):
            chunk = core * cpc + j
            plsc.subcore_barrier()
            pref(chunk, 0, 0)
            pref(chunk, 1, 1)

            @pl.loop(0, nbt, step=nbuf)
            def _(b0):
                for kb in range(nbuf):
                    bi_ = b0 + kb
                    ot = (kb + 2) % nbuf
                    wait_pay(kb)
                    pltpu.async_copy(pbufs[kb], acc.at[idx_v.at[bi_]],
                                     ssems[kb], add=True)

                    @pl.when(bi_ >= 2)
                    def _():
                        wait_scat(ot)

                    @pl.when(bi_ + 2 < nbt)
                    def _():
                        pref(chunk, bi_ + 2, ot)

            wait_scat((nbt - 2) % nbuf)
            wait_scat((nbt - 1) % nbuf)
            plsc.subcore_barrier()
            pltpu.sync_copy(acc.at[pl.ds(sbase, stripe)],
                            out_hbm.at[chunk, pl.ds(sbase, stripe)])
            if j < cpc - 1:
                zero_stripe()

    return k(payload, row3)


def _node_tc(moments, moments2, ln_g, ln_b, mlp1_w, mlp1_b, mlp2_wr, mlp2_br,
             lini_w, lina_w, lins_w, nb, n):
    h = moments.shape[2]

    def body(m_ref, m2_ref, g_ref, be_ref, w1_ref, b1_ref, w2_ref, b2_ref,
             li_ref, la_ref, ls_ref, out_ref):
        m = m_ref[...] + m2_ref[...]
        r = m[0]
        p0, p1, p2 = m[1], m[2], m[3]
        q0, q1, q2, q3, q4, q5 = (m[4], m[5], m[6], m[7], m[8], m[9])
        tn = ((r + q0) ** 2 + (r + q3) ** 2 + (r + q5) ** 2
              + 2.0 * (q1 * q1 + p2 * p2)
              + 2.0 * (q2 * q2 + p1 * p1)
              + 2.0 * (q4 * q4 + p0 * p0))
        mu = jnp.mean(tn, axis=1, keepdims=True)
        var = jnp.mean((tn - mu) ** 2, axis=1, keepdims=True)
        hh = (tn - mu) * lax.rsqrt(var + 1e-5) * g_ref[...] + be_ref[...]
        h1 = jnp.dot(hh, w1_ref[...], preferred_element_type=jnp.float32) + b1_ref[...]
        h1 = h1 / (1.0 + jnp.exp(-h1))
        h2 = jnp.dot(h1, w2_ref[...], preferred_element_type=jnp.float32) + b2_ref[...]
        n0 = h2[:, :h]
        n1 = h2[:, h:2 * h]
        n2 = h2[:, 2 * h:]
        dot = lambda x, wref: jnp.dot(x, wref[...], preferred_element_type=jnp.float32)
        ni = dot(r, li_ref) * n0
        np0 = dot(p0, la_ref) * n1
        np1 = dot(p1, la_ref) * n1
        np2 = dot(p2, la_ref) * n1
        nq0 = dot(q0, ls_ref) * n2
        nq1 = dot(q1, ls_ref) * n2
        nq2 = dot(q2, ls_ref) * n2
        nq3 = dot(q3, ls_ref) * n2
        nq4 = dot(q4, ls_ref) * n2
        nq5 = dot(q5, ls_ref) * n2
        out_ref[0] = ni + nq0
        out_ref[1] = nq1 - np2
        out_ref[2] = nq2 + np1
        out_ref[3] = nq1 + np2
        out_ref[4] = ni + nq3
        out_ref[5] = nq4 - np0
        out_ref[6] = nq2 - np1
        out_ref[7] = nq4 + np0
        out_ref[8] = ni + nq5

    full = lambda shape: pl.BlockSpec(shape, lambda i: tuple(0 for _ in shape))
    return pl.pallas_call(
        body,
        grid=(n // nb,),
        in_specs=[
            pl.BlockSpec((10, nb, h), lambda i: (0, i, 0)),
            pl.BlockSpec((10, nb, h), lambda i: (0, i, 0)),
            full((1, h)), full((1, h)),
            full(mlp1_w.shape), full((1, mlp1_w.shape[1])),
            full(mlp2_wr.shape), full((1, mlp2_wr.shape[1])),
            full((h, h)), full((h, h)), full((h, h)),
        ],
        out_specs=pl.BlockSpec((9, nb, h), lambda i: (0, i, 0)),
        out_shape=jax.ShapeDtypeStruct((9, n, h), jnp.float32),
        compiler_params=pltpu.CompilerParams(
            dimension_semantics=("arbitrary",)),
    )(moments, moments2, ln_g, ln_b, mlp1_w, mlp1_b, mlp2_wr, mlp2_br,
      lini_w, lina_w, lins_w)


def kernel(z, edge_index, edge_dist, edge_vec_norm, edge_attr,
           atom_emb, atom_proj_w, atom_proj_b,
           distI_w, distI_b, distA_w, distA_b, distS_w, distS_b,
           linI_w, linA_w, linS_w, ln_g, ln_b,
           mlp1_w, mlp1_b, mlp2_w, mlp2_b):
    n = z.shape[0]
    e = edge_index.shape[1]
    h = atom_emb.shape[1]

    z = z.astype(jnp.int32)

    e_pad = ((e + 4095) // 4096) * 4096
    pe = e_pad - e
    row_i = edge_index[0].astype(jnp.int32)
    col_i = edge_index[1].astype(jnp.int32)
    row_p = jnp.pad(row_i, (0, pe))
    col_p = jnp.pad(col_i, (0, pe))
    row_scat = jnp.pad(row_i, (0, pe), constant_values=n)
    eidx = jnp.concatenate([row_p, col_p])

    zflat = _gather_z_sc(z, eidx)

    emb_p = jnp.pad(atom_emb, ((0, h - atom_emb.shape[0]), (0, 0)))
    scal = jnp.concatenate([
        zflat.reshape(2, e_pad).astype(jnp.float32),
        jnp.pad(jnp.stack([edge_dist, edge_vec_norm[:, 0],
                           edge_vec_norm[:, 1], edge_vec_norm[:, 2]]),
                ((0, 0), (0, pe))),
    ])
    row1 = lambda x: x.reshape(1, -1)

    b = 64
    eb = 1280
    n_pad = ((n + 2047) // 2048) * 2048
    e_half = e_pad // 2
    moments = []
    for s in range(2):
        sl = slice(s * e_half, (s + 1) * e_half)
        pay = _edge_payload_tc(
            scal, edge_attr, emb_p, atom_proj_w, row1(atom_proj_b),
            distI_w, row1(distI_b), distA_w, row1(distA_b),
            distS_w, row1(distS_b), eb=eb,
            off_blk=s * (e_half // eb), nblk=e_half // eb)
        row3 = row_scat[sl].reshape(16, e_half // 16 // b, b)
        moments.append(_scatter_sc(pay, row3, n_pad))

    perm = (3 * np.arange(h)[None, :] + np.arange(3)[:, None]).reshape(-1)
    mlp2_wr = mlp2_w[:, perm]
    mlp2_br = mlp2_b[perm].reshape(1, -1)

    out9 = _node_tc(moments[0], moments[1], row1(ln_g), row1(ln_b),
                    mlp1_w, row1(mlp1_b), mlp2_wr, mlp2_br,
                    linI_w, linA_w, linS_w, nb=1000, n=n)
    return jnp.transpose(out9, (1, 2, 0)).reshape(n, h, 3, 3)


# speedup vs baseline: 1.1172x; 1.0820x over previous
import functools
import math

import jax
import jax.numpy as jnp
import numpy as np
from jax import lax
from jax.experimental import pallas as pl
from jax.experimental.pallas import tpu as pltpu
from jax.experimental.pallas import tpu_sc as plsc

CUTOFF = 5.0


def _gather_z_sc(z, eidx_flat):
    n = z.shape[0]
    te = eidx_flat.shape[0]
    nw = 32
    per = te // nw
    iters = per // 16
    assert per * nw == te and iters * 16 == per

    mesh = plsc.VectorSubcoreMesh(core_axis_name="c", subcore_axis_name="s")

    @functools.partial(
        pl.kernel,
        out_type=jax.ShapeDtypeStruct((te,), jnp.int32),
        mesh=mesh,
        compiler_params=pltpu.CompilerParams(needs_layout_passes=False),
        scratch_types=[
            pltpu.VMEM((n,), jnp.int32),
            pltpu.VMEM((per,), jnp.int32),
            pltpu.VMEM((per,), jnp.int32),
        ],
    )
    def k(z_hbm, idx_hbm, out_hbm, z_v, idx_v, out_v):
        wid = lax.axis_index("s") * 2 + lax.axis_index("c")
        base = wid * per
        pltpu.sync_copy(z_hbm, z_v)
        pltpu.sync_copy(idx_hbm.at[pl.ds(base, per)], idx_v)

        @pl.loop(0, iters)
        def _(i):
            idx = idx_v[pl.ds(i * 16, 16)]
            out_v[pl.ds(i * 16, 16)] = plsc.load_gather(z_v, [idx])

        pltpu.sync_copy(out_v, out_hbm.at[pl.ds(base, per)])

    return k(z, eidx_flat)


def _edge_payload_tc(scal, edge_attr, emb_p, apw, apb,
                     wi, bi, wa, ba, ws, bs, eb, off_blk, nblk):
    e = edge_attr.shape[0]
    h = apw.shape[1]
    nrbf = edge_attr.shape[1]
    assert e % eb == 0
    last_ea_blk = e // eb - 1
    dn = (((0,), (0,)), ((), ()))

    def body(sc_ref, ea_ref, emb_ref, apw_ref, apb_ref,
             wi_ref, bi_ref, wa_ref, ba_ref, ws_ref, bs_ref, out_ref):
        f32 = jnp.float32
        a1 = jnp.dot(emb_ref[...], apw_ref[:h, :], preferred_element_type=f32)
        a2 = jnp.dot(emb_ref[...], apw_ref[h:, :], preferred_element_type=f32)
        ones_r = jnp.ones((1, h), f32)
        s = sc_ref[...]
        bc = lambda r: lax.dot_general(r, ones_r, dn,
                                       preferred_element_type=f32)
        bch = lambda r: lax.dot_general(r, ones_r, dn,
                                        preferred_element_type=f32,
                                        precision=lax.Precision.HIGHEST)
        zrm = bc(s[0:1, :])
        zcm = bc(s[1:2, :])
        cols = lax.broadcasted_iota(jnp.int32, (eb, h), 1).astype(f32)
        ohr = (zrm == cols).astype(f32)
        ohc = (zcm == cols).astype(f32)
        zij = (jnp.dot(ohr, a1, preferred_element_type=f32)
               + jnp.dot(ohc, a2, preferred_element_type=f32)
               + apb_ref[...])
        d = s[2:3, :]
        cval = 0.5 * (jnp.cos(d * (math.pi / CUTOFF)) + 1.0)
        cval = jnp.where(d < CUTOFF, cval, 0.0)
        czm = bch(cval) * zij
        ea = ea_ref[...]
        di = jnp.dot(ea, wi_ref[...], preferred_element_type=f32) + bi_ref[...]
        da = jnp.dot(ea, wa_ref[...], preferred_element_type=f32) + ba_ref[...]
        ds_ = jnp.dot(ea, ws_ref[...], preferred_element_type=f32) + bs_ref[...]
        w_i = di * czm
        w_a = da * czm
        w_s = ds_ * czm
        a0 = bch(s[3:4, :])
        a1v = bch(s[4:5, :])
        a2v = bch(s[5:6, :])
        out_ref[0] = w_i
        out_ref[1] = w_a * a0
        out_ref[2] = w_a * a1v
        out_ref[3] = w_a * a2v
        out_ref[4] = w_s * (a0 * a0)
        out_ref[5] = w_s * (a0 * a1v)
        out_ref[6] = w_s * (a0 * a2v)
        out_ref[7] = w_s * (a1v * a1v)
        out_ref[8] = w_s * (a1v * a2v)
        out_ref[9] = w_s * (a2v * a2v)

    full = lambda shape: pl.BlockSpec(shape, lambda i: tuple(0 for _ in shape))
    return pl.pallas_call(
        body,
        grid=(nblk,),
        in_specs=[
            pl.BlockSpec((6, eb), lambda i: (0, i + off_blk)),
            pl.BlockSpec((eb, nrbf),
                         lambda i: (jnp.minimum(i + off_blk, last_ea_blk), 0)),
            full(emb_p.shape), full(apw.shape), full((1, h)),
            full((nrbf, h)), full((1, h)),
            full((nrbf, h)), full((1, h)),
            full((nrbf, h)), full((1, h)),
        ],
        out_specs=pl.BlockSpec((10, eb, h), lambda i: (0, i, 0)),
        out_shape=jax.ShapeDtypeStruct((10, nblk * eb, h), jnp.float32),
        compiler_params=pltpu.CompilerParams(
            dimension_semantics=("arbitrary",)),
    )(scal, edge_attr, emb_p, apw, apb, wi, bi, wa, ba, ws, bs)


def _scatter_sc(payload, row3, n, init=None):
    c, e, h = payload.shape
    nt = 16
    per_tile = e // nt
    nbt, b = row3.shape[1], row3.shape[2]
    assert nbt * b == per_tile and row3.shape[0] == nt
    stripe = n // nt
    zr = 16
    assert stripe % zr == 0 and stripe % 8 == 0
    cpc = c // 2
    nbuf = 2
    assert nbt % nbuf == 0
    with_init = init is not None

    mesh = plsc.VectorSubcoreMesh(core_axis_name="c", subcore_axis_name="s")

    @functools.partial(
        pl.kernel,
        out_type=jax.ShapeDtypeStruct((c, n, h), jnp.float32),
        mesh=mesh,
        compiler_params=pltpu.CompilerParams(needs_layout_passes=False),
        scratch_types=(
            [pltpu.VMEM_SHARED((n, h), jnp.float32)]
            + [pltpu.VMEM((b, h), jnp.float32) for _ in range(nbuf)]
            + [pltpu.VMEM((nbt, b), jnp.int32),
               pltpu.VMEM((zr, h), jnp.float32)]
            + [pltpu.SemaphoreType.DMA for _ in range(2 * nbuf + 1)]
        ),
    )
    def k(*refs):
        if with_init:
            pay_hbm, row_hbm, init_hbm, out_hbm = refs[:4]
            scr = refs[4:]
        else:
            pay_hbm, row_hbm, out_hbm = refs[:3]
            scr = refs[3:]
        acc, pb0, pb1, idx_v, zb, sp0, sp1, ss0, ss1, zs = scr
        core = lax.axis_index("c")
        sub = lax.axis_index("s")
        pbufs = [pb0, pb1]
        psems = [sp0, sp1]
        ssems = [ss0, ss1]

        pltpu.sync_copy(row_hbm.at[sub], idx_v)

        @pl.loop(0, zr)
        def _(r):
            for c16 in range(h // 16):
                zb[r, pl.ds(c16 * 16, 16)] = jnp.zeros((16,), jnp.float32)

        sbase = sub * stripe
        nz = stripe // zr

        def init_stripe(chunk):
            if with_init:
                pltpu.async_copy(init_hbm.at[chunk, pl.ds(sbase, stripe)],
                                 acc.at[pl.ds(sbase, stripe)], zs)
                pltpu.make_async_copy(
                    init_hbm.at[chunk, pl.ds(sbase, stripe)],
                    acc.at[pl.ds(sbase, stripe)], zs).wait()
            else:
                @pl.loop(0, nz)
                def _(t):
                    pltpu.async_copy(zb, acc.at[pl.ds(sbase + t * zr, zr)],
                                     zs)

                @pl.loop(0, nz)
                def _(t):
                    pltpu.make_async_copy(
                        zb, acc.at[pl.ds(sbase, zr)], zs).wait()

        def wait_scat(kb):
            pltpu.make_async_copy(
                pbufs[kb], acc.at[idx_v.at[0]], ssems[kb]).wait()

        init_stripe(core * cpc)
        for j in range(cpc):
            chunk = core * cpc + j
            plsc.subcore_barrier()
            for kb in range(nbuf):
                base = sub * per_tile + kb * b
                pltpu.async_copy(pay_hbm.at[chunk, pl.ds(base, b)],
                                 pbufs[kb], psems[kb])

            @pl.loop(0, nbt, step=nbuf)
            def _(b0):
                for kb in range(nbuf):
                    bi_ = b0 + kb
                    pltpu.make_async_copy(
                        pay_hbm.at[0, pl.ds(0, b)], pbufs[kb],
                        psems[kb]).wait()
                    pltpu.async_copy(pbufs[kb], acc.at[idx_v.at[bi_]],
                                     ssems[kb], add=True)
                    nb = bi_ + nbuf

                    @pl.when(nb < nbt)
                    def _():
                        wait_scat(kb)
                        base = sub * per_tile + nb * b
                        pltpu.async_copy(pay_hbm.at[chunk, pl.ds(base, b)],
                                         pbufs[kb], psems[kb])

            for kb in range(nbuf):
                wait_scat(kb)
            plsc.subcore_barrier()
            pltpu.sync_copy(acc.at[pl.ds(sbase, stripe)],
                            out_hbm.at[chunk, pl.ds(sbase, stripe)])
            if j < cpc - 1:
                init_stripe(core * cpc + j + 1)

    if with_init:
        return k(payload, row3, init)
    return k(payload, row3)


def _node_tc(moments, ln_g, ln_b, mlp1_w, mlp1_b, mlp2_wr, mlp2_br,
             lini_w, lina_w, lins_w, nb, n):
    h = moments.shape[2]

    def body(m_ref, g_ref, be_ref, w1_ref, b1_ref, w2_ref, b2_ref,
             li_ref, la_ref, ls_ref, out_ref):
        r = m_ref[0]
        p0, p1, p2 = m_ref[1], m_ref[2], m_ref[3]
        q0, q1, q2, q3, q4, q5 = (m_ref[4], m_ref[5], m_ref[6],
                                  m_ref[7], m_ref[8], m_ref[9])
        tn = ((r + q0) ** 2 + (r + q3) ** 2 + (r + q5) ** 2
              + 2.0 * (q1 * q1 + p2 * p2)
              + 2.0 * (q2 * q2 + p1 * p1)
              + 2.0 * (q4 * q4 + p0 * p0))
        mu = jnp.mean(tn, axis=1, keepdims=True)
        var = jnp.mean((tn - mu) ** 2, axis=1, keepdims=True)
        hh = (tn - mu) * lax.rsqrt(var + 1e-5) * g_ref[...] + be_ref[...]
        h1 = jnp.dot(hh, w1_ref[...], preferred_element_type=jnp.float32) + b1_ref[...]
        h1 = h1 / (1.0 + jnp.exp(-h1))
        h2 = jnp.dot(h1, w2_ref[...], preferred_element_type=jnp.float32) + b2_ref[...]
        n0 = h2[:, :h]
        n1 = h2[:, h:2 * h]
        n2 = h2[:, 2 * h:]
        dot = lambda x, wref: jnp.dot(x, wref[...], preferred_element_type=jnp.float32)
        ni = dot(r, li_ref) * n0
        np0 = dot(p0, la_ref) * n1
        np1 = dot(p1, la_ref) * n1
        np2 = dot(p2, la_ref) * n1
        nq0 = dot(q0, ls_ref) * n2
        nq1 = dot(q1, ls_ref) * n2
        nq2 = dot(q2, ls_ref) * n2
        nq3 = dot(q3, ls_ref) * n2
        nq4 = dot(q4, ls_ref) * n2
        nq5 = dot(q5, ls_ref) * n2
        out_ref[0] = ni + nq0
        out_ref[1] = nq1 - np2
        out_ref[2] = nq2 + np1
        out_ref[3] = nq1 + np2
        out_ref[4] = ni + nq3
        out_ref[5] = nq4 - np0
        out_ref[6] = nq2 - np1
        out_ref[7] = nq4 + np0
        out_ref[8] = ni + nq5

    full = lambda shape: pl.BlockSpec(shape, lambda i: tuple(0 for _ in shape))
    return pl.pallas_call(
        body,
        grid=(n // nb,),
        in_specs=[
            pl.BlockSpec((10, nb, h), lambda i: (0, i, 0)),
            full((1, h)), full((1, h)),
            full(mlp1_w.shape), full((1, mlp1_w.shape[1])),
            full(mlp2_wr.shape), full((1, mlp2_wr.shape[1])),
            full((h, h)), full((h, h)), full((h, h)),
        ],
        out_specs=pl.BlockSpec((9, nb, h), lambda i: (0, i, 0)),
        out_shape=jax.ShapeDtypeStruct((9, n, h), jnp.float32),
        compiler_params=pltpu.CompilerParams(
            dimension_semantics=("arbitrary",)),
    )(moments, ln_g, ln_b, mlp1_w, mlp1_b, mlp2_wr, mlp2_br,
      lini_w, lina_w, lins_w)


def kernel(z, edge_index, edge_dist, edge_vec_norm, edge_attr,
           atom_emb, atom_proj_w, atom_proj_b,
           distI_w, distI_b, distA_w, distA_b, distS_w, distS_b,
           linI_w, linA_w, linS_w, ln_g, ln_b,
           mlp1_w, mlp1_b, mlp2_w, mlp2_b):
    n = z.shape[0]
    e = edge_index.shape[1]
    h = atom_emb.shape[1]

    z = z.astype(jnp.int32)

    e_pad = ((e + 4095) // 4096) * 4096
    pe = e_pad - e
    row_i = edge_index[0].astype(jnp.int32)
    col_i = edge_index[1].astype(jnp.int32)
    row_p = jnp.pad(row_i, (0, pe))
    col_p = jnp.pad(col_i, (0, pe))
    row_scat = jnp.pad(row_i, (0, pe), constant_values=n)
    eidx = jnp.concatenate([row_p, col_p])

    zflat = _gather_z_sc(z, eidx)

    emb_p = jnp.pad(atom_emb, ((0, h - atom_emb.shape[0]), (0, 0)))
    scal = jnp.concatenate([
        zflat.reshape(2, e_pad).astype(jnp.float32),
        jnp.pad(jnp.stack([edge_dist, edge_vec_norm[:, 0],
                           edge_vec_norm[:, 1], edge_vec_norm[:, 2]]),
                ((0, 0), (0, pe))),
    ])
    row1 = lambda x: x.reshape(1, -1)

    b = 128
    eb = 1280
    n_pad = ((n + 2047) // 2048) * 2048
    e_half = e_pad // 2
    moments = None
    for s in range(2):
        sl = slice(s * e_half, (s + 1) * e_half)
        pay = _edge_payload_tc(
            scal, edge_attr, emb_p, atom_proj_w, row1(atom_proj_b),
            distI_w, row1(distI_b), distA_w, row1(distA_b),
            distS_w, row1(distS_b), eb=eb,
            off_blk=s * (e_half // eb), nblk=e_half // eb)
        row3 = row_scat[sl].reshape(16, e_half // 16 // b, b)
        moments = _scatter_sc(pay, row3, n_pad, init=moments)

    perm = (3 * np.arange(h)[None, :] + np.arange(3)[:, None]).reshape(-1)
    mlp2_wr = mlp2_w[:, perm]
    mlp2_br = mlp2_b[perm].reshape(1, -1)

    out9 = _node_tc(moments, row1(ln_g), row1(ln_b),
                    mlp1_w, row1(mlp1_b), mlp2_wr, mlp2_br,
                    linI_w, linA_w, linS_w, nb=1000, n=n)
    return jnp.transpose(out9, (1, 2, 0)).reshape(n, h, 3, 3)
